# baseline (speedup 1.0000x reference)
import jax
import jax.numpy as jnp
from jax.experimental import pallas as pl
from jax.experimental.pallas import tpu as pltpu

_TOKENS = 128
_HIDDEN = 1024
_INTER = 512
_NE = 64
_TOPK = 8
_NG = 8
_TOPKG = 3
_EPB = 4


def _routing(x, gw):
    logits = jax.lax.dot_general(
        x, gw, (((1,), (1,)), ((), ())), preferred_element_type=jnp.float32)
    m = jnp.max(logits, axis=-1, keepdims=True)
    ex = jnp.exp(logits - m)
    probs = ex / jnp.sum(ex, axis=-1, keepdims=True)

    gsize = _NE // _NG
    gs = jnp.concatenate(
        [jnp.max(probs[:, g * gsize:(g + 1) * gsize], axis=-1, keepdims=True)
         for g in range(_NG)],
        axis=-1)

    iota_g = jax.lax.broadcasted_iota(jnp.int32, (_TOKENS, _NG), 1)
    gmask = jnp.zeros((_TOKENS, _NG), jnp.float32)
    gwork = gs
    for _ in range(_TOPKG):
        mx = jnp.max(gwork, axis=-1, keepdims=True)
        idx = jnp.min(jnp.where(gwork == mx, iota_g, _NG), axis=-1,
                      keepdims=True)
        sel = iota_g == idx
        gmask = gmask + jnp.where(sel, 1.0, 0.0)
        gwork = jnp.where(sel, -jnp.inf, gwork)

    ig_r = jax.lax.broadcasted_iota(jnp.int32, (_NG, _NE), 0)
    ig_c = jax.lax.broadcasted_iota(jnp.int32, (_NG, _NE), 1)
    member = jnp.where(ig_r == ig_c // gsize, 1.0, 0.0)
    emask = jax.lax.dot_general(
        gmask, member, (((1,), (0,)), ((), ())),
        preferred_element_type=jnp.float32)

    ts = jnp.where(emask > 0, probs, 0.0)
    iota_e = jax.lax.broadcasted_iota(jnp.int32, (_TOKENS, _NE), 1)
    dw = jnp.zeros((_TOKENS, _NE), jnp.float32)
    for _ in range(_TOPK):
        mx = jnp.max(ts, axis=-1, keepdims=True)
        idx = jnp.min(jnp.where(ts == mx, iota_e, _NE), axis=-1,
                      keepdims=True)
        sel = iota_e == idx
        dw = dw + jnp.where(sel, ts, 0.0)
        ts = jnp.where(sel, -1.0, ts)

    return dw.T


def _moe_kernel(x_ref, gw_ref, w1_ref, w3_ref, w2_ref, out_ref, dwt_s):
    i = pl.program_id(0)

    @pl.when(i == 0)
    def _():
        dwt_s[...] = _routing(x_ref[...], gw_ref[...])
        out_ref[...] = jnp.zeros_like(out_ref)

    x = x_ref[...]
    ir = jax.lax.broadcasted_iota(jnp.int32, (_TOKENS, _TOKENS), 0)
    ic = jax.lax.broadcasted_iota(jnp.int32, (_TOKENS, _TOKENS), 1)

    acc = jnp.zeros((_TOKENS, _HIDDEN), jnp.float32)
    for j in range(_EPB):
        w1 = w1_ref[j]
        w3 = w3_ref[j]
        w2 = w2_ref[j]
        h1 = jax.lax.dot_general(
            x, w1, (((1,), (1,)), ((), ())),
            preferred_element_type=jnp.float32)
        h3 = jax.lax.dot_general(
            x, w3, (((1,), (1,)), ((), ())),
            preferred_element_type=jnp.float32)
        h = h1 * jax.lax.logistic(h1) * h3

        wrow = dwt_s[pl.ds(i * _EPB + j, 1), :]
        wb = jnp.broadcast_to(wrow, (_TOKENS, _TOKENS))
        dmat = jnp.where(ir == ic, wb, 0.0)
        hw = jax.lax.dot_general(
            dmat, h, (((1,), (0,)), ((), ())),
            preferred_element_type=jnp.float32)
        acc = acc + jax.lax.dot_general(
            hw, w2, (((1,), (1,)), ((), ())),
            preferred_element_type=jnp.float32)

    out_ref[...] += acc


def kernel(hidden_states, gate_w, w1, w2, w3):
    out = pl.pallas_call(
        _moe_kernel,
        grid=(_NE // _EPB,),
        in_specs=[
            pl.BlockSpec((_TOKENS, _HIDDEN), lambda e: (0, 0)),
            pl.BlockSpec((_NE, _HIDDEN), lambda e: (0, 0)),
            pl.BlockSpec((_EPB, _INTER, _HIDDEN), lambda e: (e, 0, 0)),
            pl.BlockSpec((_EPB, _INTER, _HIDDEN), lambda e: (e, 0, 0)),
            pl.BlockSpec((_EPB, _HIDDEN, _INTER), lambda e: (e, 0, 0)),
        ],
        out_specs=pl.BlockSpec((_TOKENS, _HIDDEN), lambda e: (0, 0)),
        out_shape=jax.ShapeDtypeStruct((_TOKENS, _HIDDEN), jnp.float32),
        scratch_shapes=[pltpu.VMEM((_NE, _TOKENS), jnp.float32)],
    )(hidden_states, gate_w, w1, w3, w2)
    return out


# speedup vs baseline: 1.0304x; 1.0304x over previous
import jax
import jax.numpy as jnp
from jax.experimental import pallas as pl
from jax.experimental.pallas import tpu as pltpu

_TOKENS = 128
_HIDDEN = 1024
_INTER = 512
_NE = 64
_TOPK = 8
_NG = 8
_TOPKG = 3


def _routing(x, gw):
    logits = jax.lax.dot_general(
        x, gw, (((1,), (1,)), ((), ())), preferred_element_type=jnp.float32)
    m = jnp.max(logits, axis=-1, keepdims=True)
    ex = jnp.exp(logits - m)
    probs = ex / jnp.sum(ex, axis=-1, keepdims=True)

    gsize = _NE // _NG
    gs = jnp.concatenate(
        [jnp.max(probs[:, g * gsize:(g + 1) * gsize], axis=-1, keepdims=True)
         for g in range(_NG)],
        axis=-1)

    iota_g = jax.lax.broadcasted_iota(jnp.int32, (_TOKENS, _NG), 1)
    gmask = jnp.zeros((_TOKENS, _NG), jnp.float32)
    gwork = gs
    for _ in range(_TOPKG):
        mx = jnp.max(gwork, axis=-1, keepdims=True)
        idx = jnp.min(jnp.where(gwork == mx, iota_g, _NG), axis=-1,
                      keepdims=True)
        sel = iota_g == idx
        gmask = gmask + jnp.where(sel, 1.0, 0.0)
        gwork = jnp.where(sel, -jnp.inf, gwork)

    ig_r = jax.lax.broadcasted_iota(jnp.int32, (_NG, _NE), 0)
    ig_c = jax.lax.broadcasted_iota(jnp.int32, (_NG, _NE), 1)
    member = jnp.where(ig_r == ig_c // gsize, 1.0, 0.0)
    emask = jax.lax.dot_general(
        gmask, member, (((1,), (0,)), ((), ())),
        preferred_element_type=jnp.float32)

    ts = jnp.where(emask > 0, probs, 0.0)
    iota_e = jax.lax.broadcasted_iota(jnp.int32, (_TOKENS, _NE), 1)
    dw = jnp.zeros((_TOKENS, _NE), jnp.float32)
    for _ in range(_TOPK):
        mx = jnp.max(ts, axis=-1, keepdims=True)
        idx = jnp.min(jnp.where(ts == mx, iota_e, _NE), axis=-1,
                      keepdims=True)
        sel = iota_e == idx
        dw = dw + jnp.where(sel, ts, 0.0)
        ts = jnp.where(sel, -1.0, ts)

    return dw.T


def _moe_kernel(x_ref, gw_ref, w1_hbm, w3_hbm, w2_hbm, out_ref,
                w1_s, w3_s, w2_s, dwt_s, sems):
    def copies(k, slot):
        return (
            pltpu.make_async_copy(w1_hbm.at[k], w1_s.at[slot],
                                  sems.at[slot, 0]),
            pltpu.make_async_copy(w3_hbm.at[k], w3_s.at[slot],
                                  sems.at[slot, 1]),
            pltpu.make_async_copy(w2_hbm.at[k], w2_s.at[slot],
                                  sems.at[slot, 2]),
        )

    for s in (0, 1):
        for c in copies(s, s):
            c.start()

    dwt_s[...] = _routing(x_ref[...], gw_ref[...])
    out_ref[...] = jnp.zeros_like(out_ref)

    x = x_ref[...]
    ir = jax.lax.broadcasted_iota(jnp.int32, (_TOKENS, _TOKENS), 0)
    ic = jax.lax.broadcasted_iota(jnp.int32, (_TOKENS, _TOKENS), 1)

    def expert(k, slot):
        for c in copies(k, slot):
            c.wait()
        w1 = w1_s[slot]
        w3 = w3_s[slot]
        w2 = w2_s[slot]
        h1 = jax.lax.dot_general(
            x, w1, (((1,), (1,)), ((), ())),
            preferred_element_type=jnp.float32)
        h3 = jax.lax.dot_general(
            x, w3, (((1,), (1,)), ((), ())),
            preferred_element_type=jnp.float32)
        h = h1 * jax.lax.logistic(h1) * h3

        wrow = dwt_s[pl.ds(k, 1), :]
        wb = jnp.broadcast_to(wrow, (_TOKENS, _TOKENS))
        dmat = jnp.where(ir == ic, wb, 0.0)
        hw = jax.lax.dot_general(
            dmat, h, (((1,), (0,)), ((), ())),
            preferred_element_type=jnp.float32)
        contrib = jax.lax.dot_general(
            hw, w2, (((1,), (1,)), ((), ())),
            preferred_element_type=jnp.float32)
        out_ref[...] += contrib

        @pl.when(k + 2 < _NE)
        def _():
            for c in copies(k + 2, slot):
                c.start()

    def body(m, carry):
        expert(2 * m, 0)
        expert(2 * m + 1, 1)
        return carry

    jax.lax.fori_loop(0, _NE // 2, body, 0)


def kernel(hidden_states, gate_w, w1, w2, w3):
    out = pl.pallas_call(
        _moe_kernel,
        in_specs=[
            pl.BlockSpec(memory_space=pltpu.MemorySpace.VMEM),
            pl.BlockSpec(memory_space=pltpu.MemorySpace.VMEM),
            pl.BlockSpec(memory_space=pltpu.MemorySpace.HBM),
            pl.BlockSpec(memory_space=pltpu.MemorySpace.HBM),
            pl.BlockSpec(memory_space=pltpu.MemorySpace.HBM),
        ],
        out_specs=pl.BlockSpec(memory_space=pltpu.MemorySpace.VMEM),
        out_shape=jax.ShapeDtypeStruct((_TOKENS, _HIDDEN), jnp.float32),
        scratch_shapes=[
            pltpu.VMEM((2, _INTER, _HIDDEN), jnp.float32),
            pltpu.VMEM((2, _INTER, _HIDDEN), jnp.float32),
            pltpu.VMEM((2, _HIDDEN, _INTER), jnp.float32),
            pltpu.VMEM((_NE, _TOKENS), jnp.float32),
            pltpu.SemaphoreType.DMA((2, 3)),
        ],
    )(hidden_states, gate_w, w1, w3, w2)
    return out


# speedup vs baseline: 1.0916x; 1.0593x over previous
import jax
import jax.numpy as jnp
from jax.experimental import pallas as pl
from jax.experimental.pallas import tpu as pltpu

_TOKENS = 128
_HIDDEN = 1024
_INTER = 512
_NE = 64
_TOPK = 8
_NG = 8
_TOPKG = 3


def _routing(x, gw):
    logits = jax.lax.dot_general(
        x, gw, (((1,), (1,)), ((), ())), preferred_element_type=jnp.float32)
    m = jnp.max(logits, axis=-1, keepdims=True)
    ex = jnp.exp(logits - m)
    probs = ex / jnp.sum(ex, axis=-1, keepdims=True)

    gsize = _NE // _NG
    gs = jnp.concatenate(
        [jnp.max(probs[:, g * gsize:(g + 1) * gsize], axis=-1, keepdims=True)
         for g in range(_NG)],
        axis=-1)

    iota_g = jax.lax.broadcasted_iota(jnp.int32, (_TOKENS, _NG), 1)
    gmask = jnp.zeros((_TOKENS, _NG), jnp.float32)
    gwork = gs
    for _ in range(_TOPKG):
        mx = jnp.max(gwork, axis=-1, keepdims=True)
        idx = jnp.min(jnp.where(gwork == mx, iota_g, _NG), axis=-1,
                      keepdims=True)
        sel = iota_g == idx
        gmask = gmask + jnp.where(sel, 1.0, 0.0)
        gwork = jnp.where(sel, -jnp.inf, gwork)

    ig_r = jax.lax.broadcasted_iota(jnp.int32, (_NG, _NE), 0)
    ig_c = jax.lax.broadcasted_iota(jnp.int32, (_NG, _NE), 1)
    member = jnp.where(ig_r == ig_c // gsize, 1.0, 0.0)
    emask = jax.lax.dot_general(
        gmask, member, (((1,), (0,)), ((), ())),
        preferred_element_type=jnp.float32)

    ts = jnp.where(emask > 0, probs, 0.0)
    iota_e = jax.lax.broadcasted_iota(jnp.int32, (_TOKENS, _NE), 1)
    dw = jnp.zeros((_TOKENS, _NE), jnp.float32)
    for _ in range(_TOPK):
        mx = jnp.max(ts, axis=-1, keepdims=True)
        idx = jnp.min(jnp.where(ts == mx, iota_e, _NE), axis=-1,
                      keepdims=True)
        sel = iota_e == idx
        dw = dw + jnp.where(sel, ts, 0.0)
        ts = jnp.where(sel, -1.0, ts)

    return dw.T


def _moe_kernel(x_ref, gw_ref, w1_hbm, w3_hbm, w2_hbm, out_ref,
                w1_s, w3_s, w2_s, dwt_s, sems):
    def copies(p, slot):
        return (
            pltpu.make_async_copy(w1_hbm.at[pl.ds(2 * p, 2)], w1_s.at[slot],
                                  sems.at[slot, 0]),
            pltpu.make_async_copy(w3_hbm.at[pl.ds(2 * p, 2)], w3_s.at[slot],
                                  sems.at[slot, 1]),
            pltpu.make_async_copy(w2_hbm.at[pl.ds(2 * p, 2)], w2_s.at[slot],
                                  sems.at[slot, 2]),
        )

    for s in (0, 1):
        for c in copies(s, s):
            c.start()

    dwt_s[...] = _routing(x_ref[...], gw_ref[...])
    out_ref[...] = jnp.zeros_like(out_ref)

    x = x_ref[...]
    ir = jax.lax.broadcasted_iota(jnp.int32, (_TOKENS, _TOKENS), 0)
    ic = jax.lax.broadcasted_iota(jnp.int32, (_TOKENS, _TOKENS), 1)

    def expert(k, w1, w3, w2):
        h1 = jax.lax.dot_general(
            x, w1, (((1,), (1,)), ((), ())),
            preferred_element_type=jnp.float32)
        h3 = jax.lax.dot_general(
            x, w3, (((1,), (1,)), ((), ())),
            preferred_element_type=jnp.float32)
        h = h1 * jax.lax.logistic(h1) * h3

        wrow = dwt_s[pl.ds(k, 1), :]
        wb = jnp.broadcast_to(wrow, (_TOKENS, _TOKENS))
        dmat = jnp.where(ir == ic, wb, 0.0)
        hw = jax.lax.dot_general(
            dmat, h, (((1,), (0,)), ((), ())),
            preferred_element_type=jnp.float32)
        contrib = jax.lax.dot_general(
            hw, w2, (((1,), (1,)), ((), ())),
            preferred_element_type=jnp.float32)
        out_ref[...] += contrib

    def pair(p, slot):
        for c in copies(p, slot):
            c.wait()
        expert(2 * p, w1_s[slot, 0], w3_s[slot, 0], w2_s[slot, 0])
        expert(2 * p + 1, w1_s[slot, 1], w3_s[slot, 1], w2_s[slot, 1])

        @pl.when(p + 2 < _NE // 2)
        def _():
            for c in copies(p + 2, slot):
                c.start()

    def body(m, carry):
        pair(2 * m, 0)
        pair(2 * m + 1, 1)
        return carry

    jax.lax.fori_loop(0, _NE // 4, body, 0)


def kernel(hidden_states, gate_w, w1, w2, w3):
    out = pl.pallas_call(
        _moe_kernel,
        in_specs=[
            pl.BlockSpec(memory_space=pltpu.MemorySpace.VMEM),
            pl.BlockSpec(memory_space=pltpu.MemorySpace.VMEM),
            pl.BlockSpec(memory_space=pltpu.MemorySpace.HBM),
            pl.BlockSpec(memory_space=pltpu.MemorySpace.HBM),
            pl.BlockSpec(memory_space=pltpu.MemorySpace.HBM),
        ],
        out_specs=pl.BlockSpec(memory_space=pltpu.MemorySpace.VMEM),
        out_shape=jax.ShapeDtypeStruct((_TOKENS, _HIDDEN), jnp.float32),
        scratch_shapes=[
            pltpu.VMEM((2, 2, _INTER, _HIDDEN), jnp.float32),
            pltpu.VMEM((2, 2, _INTER, _HIDDEN), jnp.float32),
            pltpu.VMEM((2, 2, _HIDDEN, _INTER), jnp.float32),
            pltpu.VMEM((_NE, _TOKENS), jnp.float32),
            pltpu.SemaphoreType.DMA((2, 3)),
        ],
    )(hidden_states, gate_w, w1, w3, w2)
    return out
